# baseline (speedup 1.0000x reference)
import functools

import jax
import jax.numpy as jnp
from jax import lax
from jax.experimental import pallas as pl
from jax.experimental.pallas import tpu as pltpu
from jax.experimental.pallas import tpu_sc as plsc

N_NODES = 10000
N_EDGES = 320000
D = 128

N_PAD = 10240
ROWS_PER_TILE = N_PAD // 16
CHUNK = 128
N_TILES = 32
CHUNKS_PER_TILE = 80
GROUP = 8
N_GROUPS = CHUNKS_PER_TILE // GROUP
E_PAD = N_TILES * CHUNKS_PER_TILE * CHUNK

_MESH = plsc.VectorSubcoreMesh(
    core_axis_name="c", subcore_axis_name="s", num_cores=2, num_subcores=16
)


@functools.partial(
    pl.kernel,
    out_type=jax.ShapeDtypeStruct((2, N_PAD), jnp.float32),
    mesh=_MESH,
    scratch_types=[
        pltpu.VMEM((CHUNKS_PER_TILE, CHUNK), jnp.int32),
        pltpu.VMEM((CHUNK,), jnp.float32),
        pltpu.VMEM_SHARED((N_PAD,), jnp.float32),
    ],
)
def _sc_degree(dst_hbm, ones_hbm, deg_out, idx_v, ones_v, acc):
    c = lax.axis_index("c")
    s = lax.axis_index("s")
    w = c * 16 + s
    pltpu.sync_copy(ones_hbm.at[pl.ds(s * ROWS_PER_TILE, ROWS_PER_TILE)],
                    acc.at[pl.ds(s * ROWS_PER_TILE, ROWS_PER_TILE)])
    pltpu.sync_copy(dst_hbm.at[w], idx_v)
    pltpu.sync_copy(ones_hbm.at[pl.ds(0, CHUNK)], ones_v)
    plsc.subcore_barrier()

    def body(j, carry):
        pltpu.sync_copy(ones_v, acc.at[idx_v.at[j]], add=True)
        return carry

    lax.fori_loop(0, CHUNKS_PER_TILE, body, 0)
    plsc.subcore_barrier()
    pltpu.sync_copy(acc.at[pl.ds(s * ROWS_PER_TILE, ROWS_PER_TILE)],
                    deg_out.at[c, pl.ds(s * ROWS_PER_TILE, ROWS_PER_TILE)])


@functools.partial(
    pl.kernel,
    out_type=jax.ShapeDtypeStruct((2, N_PAD, D), jnp.float32),
    mesh=_MESH,
    scratch_types=[
        pltpu.VMEM((CHUNKS_PER_TILE, CHUNK), jnp.int32),
        [pltpu.VMEM((GROUP, CHUNK), jnp.int32)] * 2,
        [pltpu.VMEM((CHUNK, D), jnp.float32)] * 2,
        pltpu.VMEM_SHARED((N_PAD, D), jnp.float32),
        [pltpu.SemaphoreType.DMA] * 2,
        [pltpu.SemaphoreType.DMA] * 2,
        [pltpu.SemaphoreType.DMA] * 2,
    ],
)
def _sc_aggregate(hp_hbm, src_hbm, dst_hbm, out_hbm, sidx, dgrp, bufs, acc,
                  gsem, ssem, dsem):
    c = lax.axis_index("c")
    s = lax.axis_index("s")
    w = c * 16 + s
    pltpu.sync_copy(hp_hbm.at[pl.ds(s * ROWS_PER_TILE, ROWS_PER_TILE)],
                    acc.at[pl.ds(s * ROWS_PER_TILE, ROWS_PER_TILE)])
    pltpu.sync_copy(src_hbm.at[w], sidx)
    plsc.subcore_barrier()

    def gather(j, b):
        pltpu.async_copy(hp_hbm.at[sidx.at[j]], bufs[b], gsem[b])

    def gather_wait(j, b):
        pltpu.make_async_copy(hp_hbm.at[sidx.at[j]], bufs[b], gsem[b]).wait()

    def scatter(j, b):
        g = (j // GROUP) % 2
        pltpu.async_copy(bufs[b], acc.at[dgrp[g].at[j % GROUP]], ssem[b],
                         add=True)

    def scatter_wait(j, b):
        g = (j // GROUP) % 2
        pltpu.make_async_copy(bufs[b], acc.at[dgrp[g].at[j % GROUP]],
                              ssem[b]).wait()

    def dload(grp):
        g = grp % 2
        pltpu.async_copy(dst_hbm.at[w, pl.ds(grp * GROUP, GROUP)], dgrp[g],
                         dsem[g])

    def dload_wait(grp):
        g = grp % 2
        pltpu.make_async_copy(dst_hbm.at[w, pl.ds(grp * GROUP, GROUP)],
                              dgrp[g], dsem[g]).wait()

    dload(0)
    gather(0, 0)
    for j in range(CHUNKS_PER_TILE):
        b = j % 2
        gather_wait(j, b)
        if j > 0:
            scatter_wait(j - 1, 1 - b)
        if j % GROUP == 0:
            dload_wait(j // GROUP)
            if j // GROUP + 1 <= N_GROUPS - 1:
                dload(j // GROUP + 1)
        scatter(j, b)
        if j + 1 < CHUNKS_PER_TILE:
            gather(j + 1, 1 - b)
    scatter_wait(CHUNKS_PER_TILE - 1, (CHUNKS_PER_TILE - 1) % 2)

    plsc.subcore_barrier()
    pltpu.sync_copy(acc.at[pl.ds(s * ROWS_PER_TILE, ROWS_PER_TILE)],
                    out_hbm.at[c, pl.ds(s * ROWS_PER_TILE, ROWS_PER_TILE)])


_BLK = 632
_GRID = N_PAD // _BLK


def _tc_prescale_body(x_ref, w_ref, degp_ref, hp_ref, dis_ref):
    deg = degp_ref[0] + degp_ref[1]
    dis = lax.rsqrt(deg)
    h = jnp.dot(x_ref[...], w_ref[...], preferred_element_type=jnp.float32)
    hp_ref[...] = h * dis
    dis_ref[...] = dis


def _tc_prescale(x_pad, W1, deg_p):
    return pl.pallas_call(
        _tc_prescale_body,
        grid=(_GRID,),
        in_specs=[
            pl.BlockSpec((_BLK, D), lambda i: (i, 0)),
            pl.BlockSpec((D, D), lambda i: (0, 0)),
            pl.BlockSpec((2, _BLK, 1), lambda i: (0, i, 0)),
        ],
        out_specs=[
            pl.BlockSpec((_BLK, D), lambda i: (i, 0)),
            pl.BlockSpec((_BLK, 1), lambda i: (i, 0)),
        ],
        out_shape=[
            jax.ShapeDtypeStruct((N_PAD, D), jnp.float32),
            jax.ShapeDtypeStruct((N_PAD, 1), jnp.float32),
        ],
    )(x_pad, W1, deg_p)


def _tc_mid_body(p_ref, dis_ref, b1_ref, w2_ref, hp2_ref):
    dis = dis_ref[...]
    agg = (p_ref[0] + p_ref[1]) * dis + b1_ref[...]
    t = jnp.maximum(agg, 0.0)
    hp2_ref[...] = jnp.dot(t, w2_ref[...], preferred_element_type=jnp.float32) * dis


def _tc_mid(p, dis, b1, W2):
    return pl.pallas_call(
        _tc_mid_body,
        grid=(_GRID,),
        in_specs=[
            pl.BlockSpec((2, _BLK, D), lambda i: (0, i, 0)),
            pl.BlockSpec((_BLK, 1), lambda i: (i, 0)),
            pl.BlockSpec((1, D), lambda i: (0, 0)),
            pl.BlockSpec((D, D), lambda i: (0, 0)),
        ],
        out_specs=pl.BlockSpec((_BLK, D), lambda i: (i, 0)),
        out_shape=jax.ShapeDtypeStruct((N_PAD, D), jnp.float32),
    )(p, dis, b1, W2)


def _tc_final_body(q_ref, dis_ref, b2_ref, out_ref):
    g = (q_ref[0] + q_ref[1]) * dis_ref[...] + b2_ref[...]
    m = jnp.max(g, axis=1, keepdims=True)
    e = jnp.exp(g - m)
    lse = jnp.log(jnp.sum(e, axis=1, keepdims=True)) + m
    out_ref[...] = g - lse


def _tc_final(q, dis, b2):
    return pl.pallas_call(
        _tc_final_body,
        grid=(_GRID,),
        in_specs=[
            pl.BlockSpec((2, _BLK, D), lambda i: (0, i, 0)),
            pl.BlockSpec((_BLK, 1), lambda i: (i, 0)),
            pl.BlockSpec((1, D), lambda i: (0, 0)),
        ],
        out_specs=pl.BlockSpec((_BLK, D), lambda i: (i, 0)),
        out_shape=jax.ShapeDtypeStruct((N_PAD, D), jnp.float32),
    )(q, dis, b2)


@jax.jit
def kernel(x, edge_index, W1, b1, W2, b2):
    ei = edge_index.astype(jnp.int32)
    pad_idx = jnp.full((E_PAD - N_EDGES,), N_PAD - 1, jnp.int32)
    src3 = jnp.concatenate([ei[0], pad_idx]).reshape(N_TILES, CHUNKS_PER_TILE, CHUNK)
    dst3 = jnp.concatenate([ei[1], pad_idx]).reshape(N_TILES, CHUNKS_PER_TILE, CHUNK)

    x_pad = jnp.pad(x, ((0, N_PAD - N_NODES), (0, 0)))
    ones = jnp.ones((N_PAD,), jnp.float32)

    deg_p = _sc_degree(dst3, ones)
    hp1, dis = _tc_prescale(x_pad, W1, deg_p.reshape(2, N_PAD, 1))
    p1 = _sc_aggregate(hp1, src3, dst3)
    hp2 = _tc_mid(p1, dis, b1.reshape(1, D), W2)
    p2 = _sc_aggregate(hp2, src3, dst3)
    out = _tc_final(p2, dis, b2.reshape(1, D))
    return out[:N_NODES]


# speedup vs baseline: 1.3865x; 1.3865x over previous
import functools

import jax
import jax.numpy as jnp
from jax import lax
from jax.experimental import pallas as pl
from jax.experimental.pallas import tpu as pltpu
from jax.experimental.pallas import tpu_sc as plsc

N_NODES = 10000
N_EDGES = 320000
D = 128

N_PAD = 10240
ROWS_PER_TILE = N_PAD // 16
CHUNK = 128
C0 = 51
C1 = 107
CMAX = max(C0, C1)
E0 = 16 * C0 * CHUNK
E1 = 16 * C1 * CHUNK
E_PAD = E0 + E1

_MESH = plsc.VectorSubcoreMesh(
    core_axis_name="c", subcore_axis_name="s", num_cores=2, num_subcores=16
)


@functools.partial(
    pl.kernel,
    out_type=jax.ShapeDtypeStruct((2, N_PAD), jnp.float32),
    mesh=_MESH,
    scratch_types=[
        pltpu.VMEM((CMAX, CHUNK), jnp.int32),
        pltpu.VMEM((CHUNK,), jnp.float32),
        pltpu.VMEM_SHARED((N_PAD,), jnp.float32),
    ],
)
def _sc_degree(d0_hbm, d1_hbm, ones_hbm, deg_out, idx_v, ones_v, acc):
    c = lax.axis_index("c")
    s = lax.axis_index("s")
    pltpu.sync_copy(ones_hbm.at[pl.ds(s * ROWS_PER_TILE, ROWS_PER_TILE)],
                    acc.at[pl.ds(s * ROWS_PER_TILE, ROWS_PER_TILE)])
    pltpu.sync_copy(ones_hbm.at[pl.ds(0, CHUNK)], ones_v)

    @pl.when(c == 0)
    def _():
        pltpu.sync_copy(d0_hbm.at[s], idx_v.at[pl.ds(0, C0)])

    @pl.when(c == 1)
    def _():
        pltpu.sync_copy(d1_hbm.at[s], idx_v.at[pl.ds(0, C1)])

    plsc.subcore_barrier()
    n = jnp.where(c == 0, C0, C1)

    def body(j, carry):
        pltpu.sync_copy(ones_v, acc.at[idx_v.at[j]], add=True)
        return carry

    lax.fori_loop(0, n, body, 0)
    plsc.subcore_barrier()
    pltpu.sync_copy(acc.at[pl.ds(s * ROWS_PER_TILE, ROWS_PER_TILE)],
                    deg_out.at[c, pl.ds(s * ROWS_PER_TILE, ROWS_PER_TILE)])


@functools.partial(
    pl.kernel,
    out_type=jax.ShapeDtypeStruct((2, N_PAD, D), jnp.float32),
    mesh=_MESH,
    scratch_types=[
        pltpu.VMEM((CMAX, CHUNK), jnp.int32),
        pltpu.VMEM((CMAX, CHUNK), jnp.int32),
        pltpu.VMEM((CHUNK, D), jnp.float32),
        pltpu.VMEM_SHARED((N_PAD, D), jnp.float32),
        pltpu.SemaphoreType.DMA,
    ],
)
def _sc_aggregate(hp_hbm, s0_hbm, d0_hbm, s1_hbm, d1_hbm, out_hbm,
                  sidx, didx, buf, acc, gsem):
    c = lax.axis_index("c")
    s = lax.axis_index("s")
    pltpu.sync_copy(hp_hbm.at[pl.ds(s * ROWS_PER_TILE, ROWS_PER_TILE)],
                    acc.at[pl.ds(s * ROWS_PER_TILE, ROWS_PER_TILE)])

    @pl.when(c == 0)
    def _():
        pltpu.sync_copy(s0_hbm.at[s], sidx.at[pl.ds(0, C0)])
        pltpu.sync_copy(d0_hbm.at[s], didx.at[pl.ds(0, C0)])

    @pl.when(c == 1)
    def _():
        pltpu.sync_copy(s1_hbm.at[s], sidx.at[pl.ds(0, C1)])
        pltpu.sync_copy(d1_hbm.at[s], didx.at[pl.ds(0, C1)])

    plsc.subcore_barrier()
    n = jnp.where(c == 0, C0, C1)

    def body(j, carry):
        pltpu.async_copy(hp_hbm.at[sidx.at[j]], buf, gsem).wait()
        pltpu.sync_copy(buf, acc.at[didx.at[j]], add=True)
        return carry

    lax.fori_loop(0, n, body, 0)
    plsc.subcore_barrier()
    pltpu.sync_copy(acc.at[pl.ds(s * ROWS_PER_TILE, ROWS_PER_TILE)],
                    out_hbm.at[c, pl.ds(s * ROWS_PER_TILE, ROWS_PER_TILE)])


_BLK = 512
_GRID = N_PAD // _BLK


def _tc_prescale_body(x_ref, w_ref, degp_ref, hp_ref, dis_ref):
    deg = degp_ref[0] + degp_ref[1]
    dis = lax.rsqrt(deg)
    h = jnp.dot(x_ref[...], w_ref[...], preferred_element_type=jnp.float32)
    hp_ref[...] = h * dis
    dis_ref[...] = dis


def _tc_prescale(x_pad, W1, deg_p):
    return pl.pallas_call(
        _tc_prescale_body,
        grid=(_GRID,),
        in_specs=[
            pl.BlockSpec((_BLK, D), lambda i: (i, 0)),
            pl.BlockSpec((D, D), lambda i: (0, 0)),
            pl.BlockSpec((2, _BLK, 1), lambda i: (0, i, 0)),
        ],
        out_specs=[
            pl.BlockSpec((_BLK, D), lambda i: (i, 0)),
            pl.BlockSpec((_BLK, 1), lambda i: (i, 0)),
        ],
        out_shape=[
            jax.ShapeDtypeStruct((N_PAD, D), jnp.float32),
            jax.ShapeDtypeStruct((N_PAD, 1), jnp.float32),
        ],
    )(x_pad, W1, deg_p)


def _tc_mid_body(p_ref, dis_ref, b1_ref, w2_ref, hp2_ref):
    dis = dis_ref[...]
    agg = (p_ref[0] + p_ref[1]) * dis + b1_ref[...]
    t = jnp.maximum(agg, 0.0)
    hp2_ref[...] = jnp.dot(t, w2_ref[...], preferred_element_type=jnp.float32) * dis


def _tc_mid(p, dis, b1, W2):
    return pl.pallas_call(
        _tc_mid_body,
        grid=(_GRID,),
        in_specs=[
            pl.BlockSpec((2, _BLK, D), lambda i: (0, i, 0)),
            pl.BlockSpec((_BLK, 1), lambda i: (i, 0)),
            pl.BlockSpec((1, D), lambda i: (0, 0)),
            pl.BlockSpec((D, D), lambda i: (0, 0)),
        ],
        out_specs=pl.BlockSpec((_BLK, D), lambda i: (i, 0)),
        out_shape=jax.ShapeDtypeStruct((N_PAD, D), jnp.float32),
    )(p, dis, b1, W2)


def _tc_final_body(q_ref, dis_ref, b2_ref, out_ref):
    g = (q_ref[0] + q_ref[1]) * dis_ref[...] + b2_ref[...]
    m = jnp.max(g, axis=1, keepdims=True)
    e = jnp.exp(g - m)
    lse = jnp.log(jnp.sum(e, axis=1, keepdims=True)) + m
    out_ref[...] = g - lse


def _tc_final(q, dis, b2):
    return pl.pallas_call(
        _tc_final_body,
        grid=(_GRID,),
        in_specs=[
            pl.BlockSpec((2, _BLK, D), lambda i: (0, i, 0)),
            pl.BlockSpec((_BLK, 1), lambda i: (i, 0)),
            pl.BlockSpec((1, D), lambda i: (0, 0)),
        ],
        out_specs=pl.BlockSpec((_BLK, D), lambda i: (i, 0)),
        out_shape=jax.ShapeDtypeStruct((N_PAD, D), jnp.float32),
    )(q, dis, b2)


@jax.jit
def kernel(x, edge_index, W1, b1, W2, b2):
    ei = edge_index.astype(jnp.int32)
    pad_idx = jnp.full((E_PAD - N_EDGES,), N_PAD - 1, jnp.int32)
    src = jnp.concatenate([ei[0], pad_idx])
    dst = jnp.concatenate([ei[1], pad_idx])
    s0 = src[:E0].reshape(16, C0, CHUNK)
    d0 = dst[:E0].reshape(16, C0, CHUNK)
    s1 = src[E0:].reshape(16, C1, CHUNK)
    d1 = dst[E0:].reshape(16, C1, CHUNK)

    x_pad = jnp.pad(x, ((0, N_PAD - N_NODES), (0, 0)))
    ones = jnp.ones((N_PAD,), jnp.float32)

    deg_p = _sc_degree(d0, d1, ones)
    hp1, dis = _tc_prescale(x_pad, W1, deg_p.reshape(2, N_PAD, 1))
    p1 = _sc_aggregate(hp1, s0, d0, s1, d1)
    hp2 = _tc_mid(p1, dis, b1.reshape(1, D), W2)
    p2 = _sc_aggregate(hp2, s0, d0, s1, d1)
    out = _tc_final(p2, dis, b2.reshape(1, D))
    return out[:N_NODES]


# speedup vs baseline: 1.4843x; 1.0706x over previous
import functools

import jax
import jax.numpy as jnp
from jax import lax
from jax.experimental import pallas as pl
from jax.experimental.pallas import tpu as pltpu
from jax.experimental.pallas import tpu_sc as plsc

N_NODES = 10000
N_EDGES = 320000
D = 128

N_PAD = 10240
ROWS_PER_TILE = N_PAD // 16
CHUNK = 128
C0 = 79
C1 = 79
CMAX = max(C0, C1)
E0 = 16 * C0 * CHUNK
E1 = 16 * C1 * CHUNK
E_PAD = E0 + E1

_MESH = plsc.VectorSubcoreMesh(
    core_axis_name="c", subcore_axis_name="s", num_cores=2, num_subcores=16
)
_MESH1 = plsc.VectorSubcoreMesh(
    core_axis_name="c", subcore_axis_name="s", num_cores=1, num_subcores=16
)


@functools.partial(
    pl.kernel,
    out_type=jax.ShapeDtypeStruct((2, N_PAD), jnp.float32),
    mesh=_MESH,
    scratch_types=[
        pltpu.VMEM((CMAX, CHUNK), jnp.int32),
        pltpu.VMEM((CHUNK,), jnp.float32),
        pltpu.VMEM_SHARED((N_PAD,), jnp.float32),
    ],
)
def _sc_degree(d0_hbm, d1_hbm, ones_hbm, deg_out, idx_v, ones_v, acc):
    c = lax.axis_index("c")
    s = lax.axis_index("s")
    pltpu.sync_copy(ones_hbm.at[pl.ds(s * ROWS_PER_TILE, ROWS_PER_TILE)],
                    acc.at[pl.ds(s * ROWS_PER_TILE, ROWS_PER_TILE)])
    pltpu.sync_copy(ones_hbm.at[pl.ds(0, CHUNK)], ones_v)

    @pl.when(c == 0)
    def _():
        pltpu.sync_copy(d0_hbm.at[s], idx_v.at[pl.ds(0, C0)])

    @pl.when(c == 1)
    def _():
        pltpu.sync_copy(d1_hbm.at[s], idx_v.at[pl.ds(0, C1)])

    plsc.subcore_barrier()
    n = jnp.where(c == 0, C0, C1)

    def body(j, carry):
        pltpu.sync_copy(ones_v, acc.at[idx_v.at[j]], add=True)
        return carry

    lax.fori_loop(0, n, body, 0)
    plsc.subcore_barrier()
    pltpu.sync_copy(acc.at[pl.ds(s * ROWS_PER_TILE, ROWS_PER_TILE)],
                    deg_out.at[c, pl.ds(s * ROWS_PER_TILE, ROWS_PER_TILE)])


def _make_sc_aggregate(n_chunks):
    @functools.partial(
        pl.kernel,
        out_type=jax.ShapeDtypeStruct((N_PAD, D), jnp.float32),
        mesh=_MESH1,
        scratch_types=[
            pltpu.VMEM((n_chunks, CHUNK), jnp.int32),
            pltpu.VMEM((n_chunks, CHUNK), jnp.int32),
            pltpu.VMEM((CHUNK, D), jnp.float32),
            pltpu.VMEM_SHARED((N_PAD, D), jnp.float32),
            pltpu.SemaphoreType.DMA,
        ],
    )
    def agg(hp_hbm, src_hbm, dst_hbm, out_hbm, sidx, didx, buf, acc, gsem):
        s = lax.axis_index("s")
        pltpu.sync_copy(hp_hbm.at[pl.ds(s * ROWS_PER_TILE, ROWS_PER_TILE)],
                        acc.at[pl.ds(s * ROWS_PER_TILE, ROWS_PER_TILE)])
        pltpu.sync_copy(src_hbm.at[s], sidx)
        pltpu.sync_copy(dst_hbm.at[s], didx)
        plsc.subcore_barrier()

        def body(j, carry):
            pltpu.async_copy(hp_hbm.at[sidx.at[j]], buf, gsem).wait()
            pltpu.sync_copy(buf, acc.at[didx.at[j]], add=True)
            return carry

        lax.fori_loop(0, n_chunks, body, 0)
        plsc.subcore_barrier()
        pltpu.sync_copy(acc.at[pl.ds(s * ROWS_PER_TILE, ROWS_PER_TILE)],
                        out_hbm.at[pl.ds(s * ROWS_PER_TILE, ROWS_PER_TILE)])

    return agg


_sc_aggregate_a = _make_sc_aggregate(C0)
_sc_aggregate_b = _make_sc_aggregate(C1)


_BLK = 512
_GRID = N_PAD // _BLK


def _tc_prescale_body(x_ref, w_ref, degp_ref, hp_ref, dis_ref):
    deg = degp_ref[0] + degp_ref[1]
    dis = lax.rsqrt(deg)
    h = jnp.dot(x_ref[...], w_ref[...], preferred_element_type=jnp.float32)
    hp_ref[...] = h * dis
    dis_ref[...] = dis


def _tc_prescale(x_pad, W1, deg_p):
    return pl.pallas_call(
        _tc_prescale_body,
        grid=(_GRID,),
        in_specs=[
            pl.BlockSpec((_BLK, D), lambda i: (i, 0)),
            pl.BlockSpec((D, D), lambda i: (0, 0)),
            pl.BlockSpec((2, _BLK, 1), lambda i: (0, i, 0)),
        ],
        out_specs=[
            pl.BlockSpec((_BLK, D), lambda i: (i, 0)),
            pl.BlockSpec((_BLK, 1), lambda i: (i, 0)),
        ],
        out_shape=[
            jax.ShapeDtypeStruct((N_PAD, D), jnp.float32),
            jax.ShapeDtypeStruct((N_PAD, 1), jnp.float32),
        ],
    )(x_pad, W1, deg_p)


def _tc_mid_body(pa_ref, pb_ref, hp_ref, dis_ref, b1_ref, w2_ref, hp2_ref):
    dis = dis_ref[...]
    agg = (pa_ref[...] + pb_ref[...] - hp_ref[...]) * dis + b1_ref[...]
    t = jnp.maximum(agg, 0.0)
    hp2_ref[...] = jnp.dot(t, w2_ref[...], preferred_element_type=jnp.float32) * dis


def _tc_mid(pa, pb, hp, dis, b1, W2):
    return pl.pallas_call(
        _tc_mid_body,
        grid=(_GRID,),
        in_specs=[
            pl.BlockSpec((_BLK, D), lambda i: (i, 0)),
            pl.BlockSpec((_BLK, D), lambda i: (i, 0)),
            pl.BlockSpec((_BLK, D), lambda i: (i, 0)),
            pl.BlockSpec((_BLK, 1), lambda i: (i, 0)),
            pl.BlockSpec((1, D), lambda i: (0, 0)),
            pl.BlockSpec((D, D), lambda i: (0, 0)),
        ],
        out_specs=pl.BlockSpec((_BLK, D), lambda i: (i, 0)),
        out_shape=jax.ShapeDtypeStruct((N_PAD, D), jnp.float32),
    )(pa, pb, hp, dis, b1, W2)


def _tc_final_body(qa_ref, qb_ref, hp_ref, dis_ref, b2_ref, out_ref):
    g = (qa_ref[...] + qb_ref[...] - hp_ref[...]) * dis_ref[...] + b2_ref[...]
    m = jnp.max(g, axis=1, keepdims=True)
    e = jnp.exp(g - m)
    lse = jnp.log(jnp.sum(e, axis=1, keepdims=True)) + m
    out_ref[...] = g - lse


def _tc_final(qa, qb, hp, dis, b2):
    return pl.pallas_call(
        _tc_final_body,
        grid=(_GRID,),
        in_specs=[
            pl.BlockSpec((_BLK, D), lambda i: (i, 0)),
            pl.BlockSpec((_BLK, D), lambda i: (i, 0)),
            pl.BlockSpec((_BLK, D), lambda i: (i, 0)),
            pl.BlockSpec((_BLK, 1), lambda i: (i, 0)),
            pl.BlockSpec((1, D), lambda i: (0, 0)),
        ],
        out_specs=pl.BlockSpec((_BLK, D), lambda i: (i, 0)),
        out_shape=jax.ShapeDtypeStruct((N_PAD, D), jnp.float32),
    )(qa, qb, hp, dis, b2)


@jax.jit
def kernel(x, edge_index, W1, b1, W2, b2):
    ei = edge_index.astype(jnp.int32)
    n_pad_edges = E_PAD - N_EDGES
    pad_idx = N_NODES + jnp.arange(n_pad_edges, dtype=jnp.int32) % (N_PAD - N_NODES)
    src = jnp.concatenate([ei[0], pad_idx])
    dst = jnp.concatenate([ei[1], pad_idx])
    s0 = src[:E0].reshape(16, C0, CHUNK)
    d0 = dst[:E0].reshape(16, C0, CHUNK)
    s1 = src[E0:].reshape(16, C1, CHUNK)
    d1 = dst[E0:].reshape(16, C1, CHUNK)

    x_pad = jnp.pad(x, ((0, N_PAD - N_NODES), (0, 0)))
    ones = jnp.ones((N_PAD,), jnp.float32)

    deg_p = _sc_degree(d0, d1, ones)
    hp1, dis = _tc_prescale(x_pad, W1, deg_p.reshape(2, N_PAD, 1))
    p1a = _sc_aggregate_a(hp1, s0, d0)
    p1b = _sc_aggregate_b(hp1, s1, d1)
    hp2 = _tc_mid(p1a, p1b, hp1, dis, b1.reshape(1, D), W2)
    p2a = _sc_aggregate_a(hp2, s0, d0)
    p2b = _sc_aggregate_b(hp2, s1, d1)
    out = _tc_final(p2a, p2b, hp2, dis, b2.reshape(1, D))
    return out[:N_NODES]


# speedup vs baseline: 1.9349x; 1.3035x over previous
import functools

import jax
import jax.numpy as jnp
from jax import lax
from jax.experimental import pallas as pl
from jax.experimental.pallas import tpu as pltpu
from jax.experimental.pallas import tpu_sc as plsc

N_NODES = 10000
N_EDGES = 320000
D = 128

N_PAD = 10240
ROWS_PER_TILE = N_PAD // 16
CHUNK = 128
C0 = 80
C1 = 80
CMAX = max(C0, C1)
E0 = 16 * C0 * CHUNK
E1 = 16 * C1 * CHUNK
E_PAD = E0 + E1

_MESH = plsc.VectorSubcoreMesh(
    core_axis_name="c", subcore_axis_name="s", num_cores=2, num_subcores=16
)
_MESH1 = plsc.VectorSubcoreMesh(
    core_axis_name="c", subcore_axis_name="s", num_cores=1, num_subcores=16
)


@functools.partial(
    pl.kernel,
    out_type=jax.ShapeDtypeStruct((2, N_PAD), jnp.float32),
    mesh=_MESH,
    scratch_types=[
        pltpu.VMEM((CMAX, CHUNK), jnp.int32),
        pltpu.VMEM((CHUNK,), jnp.float32),
        pltpu.VMEM_SHARED((N_PAD,), jnp.float32),
    ],
)
def _sc_degree(d0_hbm, d1_hbm, ones_hbm, deg_out, idx_v, ones_v, acc):
    c = lax.axis_index("c")
    s = lax.axis_index("s")
    pltpu.sync_copy(ones_hbm.at[pl.ds(s * ROWS_PER_TILE, ROWS_PER_TILE)],
                    acc.at[pl.ds(s * ROWS_PER_TILE, ROWS_PER_TILE)])
    pltpu.sync_copy(ones_hbm.at[pl.ds(0, CHUNK)], ones_v)

    @pl.when(c == 0)
    def _():
        pltpu.sync_copy(d0_hbm.at[s], idx_v.at[pl.ds(0, C0)])

    @pl.when(c == 1)
    def _():
        pltpu.sync_copy(d1_hbm.at[s], idx_v.at[pl.ds(0, C1)])

    plsc.subcore_barrier()
    n = jnp.where(c == 0, C0, C1)

    def body(j, carry):
        pltpu.sync_copy(ones_v, acc.at[idx_v.at[j]], add=True)
        return carry

    lax.fori_loop(0, n, body, 0)
    plsc.subcore_barrier()
    pltpu.sync_copy(acc.at[pl.ds(s * ROWS_PER_TILE, ROWS_PER_TILE)],
                    deg_out.at[c, pl.ds(s * ROWS_PER_TILE, ROWS_PER_TILE)])


def _make_sc_aggregate(n_chunks):
    assert n_chunks % 4 == 0

    @functools.partial(
        pl.kernel,
        out_type=jax.ShapeDtypeStruct((N_PAD, D), jnp.float32),
        mesh=_MESH1,
        scratch_types=[
            pltpu.VMEM((n_chunks, CHUNK), jnp.int32),
            [pltpu.VMEM((1, CHUNK), jnp.int32)] * 4,
            [pltpu.VMEM((CHUNK, D), jnp.float32)] * 2,
            pltpu.VMEM_SHARED((N_PAD, D), jnp.float32),
            [pltpu.SemaphoreType.DMA] * 2,
            [pltpu.SemaphoreType.DMA] * 2,
            [pltpu.SemaphoreType.DMA] * 4,
        ],
    )
    def agg(hp_hbm, src_hbm, dst_hbm, out_hbm, sidx, dds, bufs, acc,
            gsem, ssem, dsem):
        s = lax.axis_index("s")
        pltpu.sync_copy(hp_hbm.at[pl.ds(s * ROWS_PER_TILE, ROWS_PER_TILE)],
                        acc.at[pl.ds(s * ROWS_PER_TILE, ROWS_PER_TILE)])
        pltpu.sync_copy(src_hbm.at[s], sidx)
        plsc.subcore_barrier()

        def gather(j, b):
            pltpu.async_copy(hp_hbm.at[sidx.at[j]], bufs[b], gsem[b])

        def gather_wait(j, b):
            pltpu.make_async_copy(hp_hbm.at[sidx.at[j]], bufs[b],
                                  gsem[b]).wait()

        def scatter(j, b, q):
            pltpu.async_copy(bufs[b], acc.at[dds[q].at[0]], ssem[b], add=True)

        def scatter_wait(j, b, q):
            pltpu.make_async_copy(bufs[b], acc.at[dds[q].at[0]],
                                  ssem[b]).wait()

        def dload(j, q):
            pltpu.async_copy(dst_hbm.at[s, pl.ds(j, 1)], dds[q], dsem[q])

        def dload_wait(j, q):
            pltpu.make_async_copy(dst_hbm.at[s, pl.ds(j, 1)], dds[q],
                                  dsem[q]).wait()

        for q in range(4):
            dload(q, q)
        gather(0, 0)
        for j in range(4):
            b, q = j % 2, j % 4
            gather_wait(j, b)
            dload_wait(j, q)
            if j > 0:
                scatter_wait(j - 1, 1 - b, (j - 1) % 4)
            scatter(j, b, q)
            if j + 2 >= 4:
                dload(j + 2, (j + 2) % 4)
            gather(j + 1, 1 - b)

        def outer(g, carry):
            jb = 4 * g
            for k in range(4):
                j = jb + k
                b, q = k % 2, k % 4
                gather_wait(j, b)
                dload_wait(j, q)
                scatter_wait(j - 1, 1 - b, (k - 1) % 4)
                scatter(j, b, q)
                dload(j + 2, (k + 2) % 4)
                gather(j + 1, 1 - b)
            return carry

        lax.fori_loop(1, n_chunks // 4 - 1, outer, 0)

        for k in range(4):
            j = n_chunks - 4 + k
            b, q = j % 2, j % 4
            gather_wait(j, b)
            dload_wait(j, q)
            scatter_wait(j - 1, 1 - b, (j - 1) % 4)
            scatter(j, b, q)
            if j + 2 < n_chunks:
                dload(j + 2, (j + 2) % 4)
            if j + 1 < n_chunks:
                gather(j + 1, 1 - b)
        scatter_wait(n_chunks - 1, (n_chunks - 1) % 2, (n_chunks - 1) % 4)

        plsc.subcore_barrier()
        pltpu.sync_copy(acc.at[pl.ds(s * ROWS_PER_TILE, ROWS_PER_TILE)],
                        out_hbm.at[pl.ds(s * ROWS_PER_TILE, ROWS_PER_TILE)])

    return agg


_sc_aggregate_a = _make_sc_aggregate(C0)
_sc_aggregate_b = _make_sc_aggregate(C1)


_BLK = 512
_GRID = N_PAD // _BLK


def _tc_prescale_body(x_ref, w_ref, degp_ref, hp_ref, dis_ref):
    deg = degp_ref[0] + degp_ref[1]
    dis = lax.rsqrt(deg)
    h = jnp.dot(x_ref[...], w_ref[...], preferred_element_type=jnp.float32)
    hp_ref[...] = h * dis
    dis_ref[...] = dis


def _tc_prescale(x_pad, W1, deg_p):
    return pl.pallas_call(
        _tc_prescale_body,
        grid=(_GRID,),
        in_specs=[
            pl.BlockSpec((_BLK, D), lambda i: (i, 0)),
            pl.BlockSpec((D, D), lambda i: (0, 0)),
            pl.BlockSpec((2, _BLK, 1), lambda i: (0, i, 0)),
        ],
        out_specs=[
            pl.BlockSpec((_BLK, D), lambda i: (i, 0)),
            pl.BlockSpec((_BLK, 1), lambda i: (i, 0)),
        ],
        out_shape=[
            jax.ShapeDtypeStruct((N_PAD, D), jnp.float32),
            jax.ShapeDtypeStruct((N_PAD, 1), jnp.float32),
        ],
    )(x_pad, W1, deg_p)


def _tc_mid_body(pa_ref, pb_ref, hp_ref, dis_ref, b1_ref, w2_ref, hp2_ref):
    dis = dis_ref[...]
    agg = (pa_ref[...] + pb_ref[...] - hp_ref[...]) * dis + b1_ref[...]
    t = jnp.maximum(agg, 0.0)
    hp2_ref[...] = jnp.dot(t, w2_ref[...], preferred_element_type=jnp.float32) * dis


def _tc_mid(pa, pb, hp, dis, b1, W2):
    return pl.pallas_call(
        _tc_mid_body,
        grid=(_GRID,),
        in_specs=[
            pl.BlockSpec((_BLK, D), lambda i: (i, 0)),
            pl.BlockSpec((_BLK, D), lambda i: (i, 0)),
            pl.BlockSpec((_BLK, D), lambda i: (i, 0)),
            pl.BlockSpec((_BLK, 1), lambda i: (i, 0)),
            pl.BlockSpec((1, D), lambda i: (0, 0)),
            pl.BlockSpec((D, D), lambda i: (0, 0)),
        ],
        out_specs=pl.BlockSpec((_BLK, D), lambda i: (i, 0)),
        out_shape=jax.ShapeDtypeStruct((N_PAD, D), jnp.float32),
    )(pa, pb, hp, dis, b1, W2)


def _tc_final_body(qa_ref, qb_ref, hp_ref, dis_ref, b2_ref, out_ref):
    g = (qa_ref[...] + qb_ref[...] - hp_ref[...]) * dis_ref[...] + b2_ref[...]
    m = jnp.max(g, axis=1, keepdims=True)
    e = jnp.exp(g - m)
    lse = jnp.log(jnp.sum(e, axis=1, keepdims=True)) + m
    out_ref[...] = g - lse


def _tc_final(qa, qb, hp, dis, b2):
    return pl.pallas_call(
        _tc_final_body,
        grid=(_GRID,),
        in_specs=[
            pl.BlockSpec((_BLK, D), lambda i: (i, 0)),
            pl.BlockSpec((_BLK, D), lambda i: (i, 0)),
            pl.BlockSpec((_BLK, D), lambda i: (i, 0)),
            pl.BlockSpec((_BLK, 1), lambda i: (i, 0)),
            pl.BlockSpec((1, D), lambda i: (0, 0)),
        ],
        out_specs=pl.BlockSpec((_BLK, D), lambda i: (i, 0)),
        out_shape=jax.ShapeDtypeStruct((N_PAD, D), jnp.float32),
    )(qa, qb, hp, dis, b2)


@jax.jit
def kernel(x, edge_index, W1, b1, W2, b2):
    ei = edge_index.astype(jnp.int32)
    n_pad_edges = E_PAD - N_EDGES
    pad_idx = N_NODES + jnp.arange(n_pad_edges, dtype=jnp.int32) % (N_PAD - N_NODES)
    src = jnp.concatenate([ei[0], pad_idx])
    dst = jnp.concatenate([ei[1], pad_idx])
    s0 = src[:E0].reshape(16, C0, CHUNK)
    d0 = dst[:E0].reshape(16, C0, CHUNK)
    s1 = src[E0:].reshape(16, C1, CHUNK)
    d1 = dst[E0:].reshape(16, C1, CHUNK)

    x_pad = jnp.pad(x, ((0, N_PAD - N_NODES), (0, 0)))
    ones = jnp.ones((N_PAD,), jnp.float32)

    deg_p = _sc_degree(d0, d1, ones)
    hp1, dis = _tc_prescale(x_pad, W1, deg_p.reshape(2, N_PAD, 1))
    p1a = _sc_aggregate_a(hp1, s0, d0)
    p1b = _sc_aggregate_b(hp1, s1, d1)
    hp2 = _tc_mid(p1a, p1b, hp1, dis, b1.reshape(1, D), W2)
    p2a = _sc_aggregate_a(hp2, s0, d0)
    p2b = _sc_aggregate_b(hp2, s1, d1)
    out = _tc_final(p2a, p2b, hp2, dis, b2.reshape(1, D))
    return out[:N_NODES]


# speedup vs baseline: 1.9359x; 1.0005x over previous
import functools

import jax
import jax.numpy as jnp
from jax import lax
from jax.experimental import pallas as pl
from jax.experimental.pallas import tpu as pltpu
from jax.experimental.pallas import tpu_sc as plsc

N_NODES = 10000
N_EDGES = 320000
D = 128

N_PAD = 10240
ROWS_PER_TILE = N_PAD // 16
CHUNK = 128
C0 = 80
C1 = 80
CMAX = max(C0, C1)
E0 = 16 * C0 * CHUNK
E1 = 16 * C1 * CHUNK
E_PAD = E0 + E1

_MESH = plsc.VectorSubcoreMesh(
    core_axis_name="c", subcore_axis_name="s", num_cores=2, num_subcores=16
)
_MESH1 = plsc.VectorSubcoreMesh(
    core_axis_name="c", subcore_axis_name="s", num_cores=1, num_subcores=16
)


@functools.partial(
    pl.kernel,
    out_type=jax.ShapeDtypeStruct((2, N_PAD), jnp.float32),
    mesh=_MESH,
    scratch_types=[
        pltpu.VMEM((CMAX, CHUNK), jnp.int32),
        pltpu.VMEM((CHUNK,), jnp.float32),
        pltpu.VMEM_SHARED((N_PAD,), jnp.float32),
    ],
)
def _sc_degree(d0_hbm, d1_hbm, ones_hbm, deg_out, idx_v, ones_v, acc):
    c = lax.axis_index("c")
    s = lax.axis_index("s")
    pltpu.sync_copy(ones_hbm.at[pl.ds(s * ROWS_PER_TILE, ROWS_PER_TILE)],
                    acc.at[pl.ds(s * ROWS_PER_TILE, ROWS_PER_TILE)])
    pltpu.sync_copy(ones_hbm.at[pl.ds(0, CHUNK)], ones_v)

    @pl.when(c == 0)
    def _():
        pltpu.sync_copy(d0_hbm.at[s], idx_v.at[pl.ds(0, C0)])

    @pl.when(c == 1)
    def _():
        pltpu.sync_copy(d1_hbm.at[s], idx_v.at[pl.ds(0, C1)])

    plsc.subcore_barrier()
    n = jnp.where(c == 0, C0, C1)

    def body(j, carry):
        pltpu.sync_copy(ones_v, acc.at[idx_v.at[j]], add=True)
        return carry

    lax.fori_loop(0, n, body, 0)
    plsc.subcore_barrier()
    pltpu.sync_copy(acc.at[pl.ds(s * ROWS_PER_TILE, ROWS_PER_TILE)],
                    deg_out.at[c, pl.ds(s * ROWS_PER_TILE, ROWS_PER_TILE)])


def _make_sc_aggregate(n_chunks):
    assert n_chunks % 4 == 0

    @functools.partial(
        pl.kernel,
        out_type=jax.ShapeDtypeStruct((N_PAD, D), jnp.float32),
        mesh=_MESH1,
        scratch_types=[
            pltpu.VMEM((n_chunks, CHUNK), jnp.int32),
            [pltpu.VMEM((1, CHUNK), jnp.int32)] * 4,
            [pltpu.VMEM((CHUNK, D), jnp.float32)] * 2,
            pltpu.VMEM_SHARED((N_PAD, D), jnp.float32),
            [pltpu.SemaphoreType.DMA] * 2,
            [pltpu.SemaphoreType.DMA] * 2,
            [pltpu.SemaphoreType.DMA] * 4,
        ],
    )
    def agg(hp_hbm, src_hbm, dst_hbm, out_hbm, sidx, dds, bufs, acc,
            gsem, ssem, dsem):
        s = lax.axis_index("s")
        pltpu.sync_copy(hp_hbm.at[pl.ds(s * ROWS_PER_TILE, ROWS_PER_TILE)],
                        acc.at[pl.ds(s * ROWS_PER_TILE, ROWS_PER_TILE)])
        pltpu.sync_copy(src_hbm.at[s], sidx)
        plsc.subcore_barrier()

        def gather(j, b):
            pltpu.async_copy(hp_hbm.at[sidx.at[j]], bufs[b], gsem[b])

        def gather_wait(j, b):
            pltpu.make_async_copy(hp_hbm.at[sidx.at[j]], bufs[b],
                                  gsem[b]).wait()

        def scatter(j, b, q):
            pltpu.async_copy(bufs[b], acc.at[dds[q].at[0]], ssem[b], add=True)

        def scatter_wait(j, b, q):
            pltpu.make_async_copy(bufs[b], acc.at[dds[q].at[0]],
                                  ssem[b]).wait()

        def dload(j, q):
            pltpu.async_copy(dst_hbm.at[s, pl.ds(j, 1)], dds[q], dsem[q])

        def dload_wait(j, q):
            pltpu.make_async_copy(dst_hbm.at[s, pl.ds(j, 1)], dds[q],
                                  dsem[q]).wait()

        for q in range(4):
            dload(q, q)
        gather(0, 0)
        for j in range(4):
            b, q = j % 2, j % 4
            gather_wait(j, b)
            dload_wait(j, q)
            if j > 0:
                scatter_wait(j - 1, 1 - b, (j - 1) % 4)
            scatter(j, b, q)
            if j + 2 >= 4:
                dload(j + 2, (j + 2) % 4)
            gather(j + 1, 1 - b)

        def outer(g, carry):
            jb = 4 * g
            for k in range(4):
                j = jb + k
                b, q = k % 2, k % 4
                gather_wait(j, b)
                dload_wait(j, q)
                scatter_wait(j - 1, 1 - b, (k - 1) % 4)
                scatter(j, b, q)
                dload(j + 2, (k + 2) % 4)
                gather(j + 1, 1 - b)
            return carry

        lax.fori_loop(1, n_chunks // 4 - 1, outer, 0)

        for k in range(4):
            j = n_chunks - 4 + k
            b, q = j % 2, j % 4
            gather_wait(j, b)
            dload_wait(j, q)
            scatter_wait(j - 1, 1 - b, (j - 1) % 4)
            scatter(j, b, q)
            if j + 2 < n_chunks:
                dload(j + 2, (j + 2) % 4)
            if j + 1 < n_chunks:
                gather(j + 1, 1 - b)
        scatter_wait(n_chunks - 1, (n_chunks - 1) % 2, (n_chunks - 1) % 4)

        plsc.subcore_barrier()
        pltpu.sync_copy(acc.at[pl.ds(s * ROWS_PER_TILE, ROWS_PER_TILE)],
                        out_hbm.at[pl.ds(s * ROWS_PER_TILE, ROWS_PER_TILE)])

    return agg


_sc_aggregate_a = _make_sc_aggregate(C0)
_sc_aggregate_b = _make_sc_aggregate(C1)


_BLK = 512
_GRID = N_PAD // _BLK


_XBLK = 400
_XGRID = N_NODES // _XBLK


def _tc_matmul_body(x_ref, w_ref, h_ref):
    h_ref[...] = jnp.dot(x_ref[...], w_ref[...],
                         preferred_element_type=jnp.float32)


def _tc_matmul(x, W1):
    return pl.pallas_call(
        _tc_matmul_body,
        grid=(_XGRID,),
        in_specs=[
            pl.BlockSpec((_XBLK, D), lambda i: (i, 0)),
            pl.BlockSpec((D, D), lambda i: (0, 0)),
        ],
        out_specs=pl.BlockSpec((_XBLK, D), lambda i: (i, 0)),
        out_shape=jax.ShapeDtypeStruct((N_PAD, D), jnp.float32),
    )(x, W1)


def _tc_prescale_body(h_ref, degp_ref, hp_ref, dis_ref):
    deg = degp_ref[0] + degp_ref[1]
    dis = lax.rsqrt(deg)
    hp_ref[...] = h_ref[...] * dis
    dis_ref[...] = dis


def _tc_prescale(h1, deg_p):
    return pl.pallas_call(
        _tc_prescale_body,
        grid=(_XGRID,),
        in_specs=[
            pl.BlockSpec((_XBLK, D), lambda i: (i, 0)),
            pl.BlockSpec((2, _XBLK, 1), lambda i: (0, i, 0)),
        ],
        out_specs=[
            pl.BlockSpec((_XBLK, D), lambda i: (i, 0)),
            pl.BlockSpec((_XBLK, 1), lambda i: (i, 0)),
        ],
        out_shape=[
            jax.ShapeDtypeStruct((N_PAD, D), jnp.float32),
            jax.ShapeDtypeStruct((N_PAD, 1), jnp.float32),
        ],
    )(h1, deg_p)


def _tc_mid_body(pa_ref, pb_ref, hp_ref, dis_ref, b1_ref, w2_ref, hp2_ref):
    dis = dis_ref[...]
    agg = (pa_ref[...] + pb_ref[...] - hp_ref[...]) * dis + b1_ref[...]
    t = jnp.maximum(agg, 0.0)
    hp2_ref[...] = jnp.dot(t, w2_ref[...], preferred_element_type=jnp.float32) * dis


def _tc_mid(pa, pb, hp, dis, b1, W2):
    return pl.pallas_call(
        _tc_mid_body,
        grid=(_GRID,),
        in_specs=[
            pl.BlockSpec((_BLK, D), lambda i: (i, 0)),
            pl.BlockSpec((_BLK, D), lambda i: (i, 0)),
            pl.BlockSpec((_BLK, D), lambda i: (i, 0)),
            pl.BlockSpec((_BLK, 1), lambda i: (i, 0)),
            pl.BlockSpec((1, D), lambda i: (0, 0)),
            pl.BlockSpec((D, D), lambda i: (0, 0)),
        ],
        out_specs=pl.BlockSpec((_BLK, D), lambda i: (i, 0)),
        out_shape=jax.ShapeDtypeStruct((N_PAD, D), jnp.float32),
    )(pa, pb, hp, dis, b1, W2)


def _tc_final_body(qa_ref, qb_ref, hp_ref, dis_ref, b2_ref, out_ref):
    g = (qa_ref[...] + qb_ref[...] - hp_ref[...]) * dis_ref[...] + b2_ref[...]
    m = jnp.max(g, axis=1, keepdims=True)
    e = jnp.exp(g - m)
    lse = jnp.log(jnp.sum(e, axis=1, keepdims=True)) + m
    out_ref[...] = g - lse


def _tc_final(qa, qb, hp, dis, b2):
    return pl.pallas_call(
        _tc_final_body,
        grid=(_GRID,),
        in_specs=[
            pl.BlockSpec((_BLK, D), lambda i: (i, 0)),
            pl.BlockSpec((_BLK, D), lambda i: (i, 0)),
            pl.BlockSpec((_BLK, D), lambda i: (i, 0)),
            pl.BlockSpec((_BLK, 1), lambda i: (i, 0)),
            pl.BlockSpec((1, D), lambda i: (0, 0)),
        ],
        out_specs=pl.BlockSpec((_BLK, D), lambda i: (i, 0)),
        out_shape=jax.ShapeDtypeStruct((N_PAD, D), jnp.float32),
    )(qa, qb, hp, dis, b2)


@jax.jit
def kernel(x, edge_index, W1, b1, W2, b2):
    ei = edge_index.astype(jnp.int32)
    n_pad_edges = E_PAD - N_EDGES
    ar = jnp.arange(n_pad_edges, dtype=jnp.int32)
    src = jnp.concatenate([ei[0], ar % N_NODES])
    dst = jnp.concatenate([ei[1], N_NODES + ar % (N_PAD - N_NODES)])
    s0 = src[:E0].reshape(16, C0, CHUNK)
    d0 = dst[:E0].reshape(16, C0, CHUNK)
    s1 = src[E0:].reshape(16, C1, CHUNK)
    d1 = dst[E0:].reshape(16, C1, CHUNK)

    ones = jnp.ones((N_PAD,), jnp.float32)

    h1 = _tc_matmul(x, W1)
    deg_p = _sc_degree(d0, d1, ones)
    hp1, dis = _tc_prescale(h1, deg_p.reshape(2, N_PAD, 1))
    p1a = _sc_aggregate_a(hp1, s0, d0)
    p1b = _sc_aggregate_b(hp1, s1, d1)
    hp2 = _tc_mid(p1a, p1b, hp1, dis, b1.reshape(1, D), W2)
    p2a = _sc_aggregate_a(hp2, s0, d0)
    p2b = _sc_aggregate_b(hp2, s1, d1)
    out = _tc_final(p2a, p2b, hp2, dis, b2.reshape(1, D))
    return out[:N_NODES]
